# baseline (device time: 62345 ns/iter reference)
import functools

import jax
import jax.numpy as jnp
from jax import lax
from jax.experimental import pallas as pl
from jax.experimental.pallas import tpu as pltpu

N_DEV = 8
BLK = 512
R = 8
G = BLK // R
N_ROUNDS = 3


def kernel(x):
    x = x.astype(jnp.float32)
    m, n = x.shape
    nblk = m // BLK

    def body(
        x_ref, o_ref, carry_ref, v_ref, comm_ref, stage_ref,
        send_sems, recv_sems,
    ):
        b = pl.program_id(0)
        i = lax.axis_index("i")

        @pl.when(b == 0)
        def _():
            carry_ref[...] = jnp.ones_like(carry_ref)

        y = x_ref[...].reshape(G, R, n)
        s = 1
        while s < R:
            pad = jnp.ones((G, s, n), jnp.float32)
            y = y * jnp.concatenate([pad, y[:, :-s, :]], axis=1)
            s *= 2
        g = y[:, R - 1, :]
        s = 1
        while s < G:
            pad = jnp.ones((s, n), jnp.float32)
            g = g * jnp.concatenate([pad, g[:-s, :]], axis=0)
            s *= 2
        c = carry_ref[...]
        e = jnp.concatenate([c, g[: G - 1, :] * c], axis=0)
        o_ref[pl.ds(b * BLK, BLK), :] = (
            (y * e[:, None, :]).astype(jnp.bfloat16).reshape(BLK, n)
        )
        carry_ref[...] = g[G - 1 : G, :] * c

        @pl.when(b == nblk - 1)
        def _():
            barrier_sem = pltpu.get_barrier_semaphore()
            for dd in range(1, N_DEV):
                pl.semaphore_signal(
                    barrier_sem, inc=1,
                    device_id=((i + dd) % N_DEV,),
                    device_id_type=pl.DeviceIdType.MESH,
                )
            pl.semaphore_wait(barrier_sem, N_DEV - 1)

            v_ref[...] = carry_ref[...]
            for r in range(N_ROUNDS):
                d = 1 << r
                send = pltpu.make_async_remote_copy(
                    src_ref=stage_ref.at[r],
                    dst_ref=comm_ref.at[r],
                    send_sem=send_sems.at[r],
                    recv_sem=recv_sems.at[r],
                    device_id=(i + d,),
                    device_id_type=pl.DeviceIdType.MESH,
                )

                @pl.when(i + d < N_DEV)
                def _():
                    stage_ref[r] = v_ref[...]
                    send.start()

                @pl.when(i >= d)
                def _():
                    send.wait_recv()
                    v_ref[...] = comm_ref[r] * v_ref[...]

                @pl.when(i + d < N_DEV)
                def _():
                    send.wait_send()

            @pl.when(i > 0)
            def _():
                p = v_ref[...] / carry_ref[...]
                o_ref[...] = (o_ref[...].astype(jnp.float32) * p).astype(
                    jnp.bfloat16
                )

            @functools.partial(
                pl.run_scoped, exit_sem=pltpu.SemaphoreType.REGULAR
            )
            def _(exit_sem):
                for dd in range(1, N_DEV):
                    pl.semaphore_signal(
                        exit_sem, inc=1,
                        device_id=((i + dd) % N_DEV,),
                        device_id_type=pl.DeviceIdType.MESH,
                    )
                pl.semaphore_wait(exit_sem, N_DEV - 1)

    return pl.pallas_call(
        body,
        grid=(nblk,),
        out_shape=jax.ShapeDtypeStruct((m, n), jnp.bfloat16),
        in_specs=[pl.BlockSpec((BLK, n), lambda b: (b, 0))],
        out_specs=pl.BlockSpec((m, n), lambda b: (0, 0)),
        scratch_shapes=[
            pltpu.VMEM((1, n), jnp.float32),
            pltpu.VMEM((1, n), jnp.float32),
            pltpu.VMEM((N_ROUNDS, 1, n), jnp.float32),
            pltpu.VMEM((N_ROUNDS, 1, n), jnp.float32),
            pltpu.SemaphoreType.DMA((N_ROUNDS,)),
            pltpu.SemaphoreType.DMA((N_ROUNDS,)),
        ],
        compiler_params=pltpu.CompilerParams(collective_id=0),
    )(x)


# device time: 45180 ns/iter; 1.3799x vs baseline; 1.3799x over previous
import functools

import jax
import jax.numpy as jnp
from jax import lax
from jax.experimental import pallas as pl
from jax.experimental.pallas import tpu as pltpu

N_DEV = 8
BLK = 512
N_ROUNDS = 3


def kernel(x):
    x = x.astype(jnp.float32)
    m, n = x.shape
    nblk = m // BLK

    def body(
        x_ref, o_ref, carry_ref, v_ref, comm_ref, stage_ref,
        send_sems, recv_sems,
    ):
        b = pl.program_id(0)
        i = lax.axis_index("i")

        @pl.when(b == 0)
        def _():
            carry_ref[...] = jnp.ones_like(carry_ref)

        y = x_ref[...]
        s = 1
        while s < BLK:
            pad = jnp.ones((s, n), jnp.float32)
            y = y * jnp.concatenate([pad, y[:-s, :]], axis=0)
            s *= 2
        y = y * carry_ref[...]
        o_ref[pl.ds(b * BLK, BLK), :] = y.astype(jnp.bfloat16)
        carry_ref[...] = y[BLK - 1 : BLK, :]

        @pl.when(b == nblk - 1)
        def _():
            barrier_sem = pltpu.get_barrier_semaphore()
            for dd in range(1, N_DEV):
                pl.semaphore_signal(
                    barrier_sem, inc=1,
                    device_id=((i + dd) % N_DEV,),
                    device_id_type=pl.DeviceIdType.MESH,
                )
            pl.semaphore_wait(barrier_sem, N_DEV - 1)

            v_ref[...] = carry_ref[...]
            for r in range(N_ROUNDS):
                d = 1 << r
                send = pltpu.make_async_remote_copy(
                    src_ref=stage_ref.at[r],
                    dst_ref=comm_ref.at[r],
                    send_sem=send_sems.at[r],
                    recv_sem=recv_sems.at[r],
                    device_id=(i + d,),
                    device_id_type=pl.DeviceIdType.MESH,
                )

                @pl.when(i + d < N_DEV)
                def _():
                    stage_ref[r] = v_ref[...]
                    send.start()

                @pl.when(i >= d)
                def _():
                    send.wait_recv()
                    v_ref[...] = comm_ref[r] * v_ref[...]

                @pl.when(i + d < N_DEV)
                def _():
                    send.wait_send()

            @pl.when(i > 0)
            def _():
                p = v_ref[...] / carry_ref[...]
                o_ref[...] = (o_ref[...].astype(jnp.float32) * p).astype(
                    jnp.bfloat16
                )

            @functools.partial(
                pl.run_scoped, exit_sem=pltpu.SemaphoreType.REGULAR
            )
            def _(exit_sem):
                for dd in range(1, N_DEV):
                    pl.semaphore_signal(
                        exit_sem, inc=1,
                        device_id=((i + dd) % N_DEV,),
                        device_id_type=pl.DeviceIdType.MESH,
                    )
                pl.semaphore_wait(exit_sem, N_DEV - 1)

    return pl.pallas_call(
        body,
        grid=(nblk,),
        out_shape=jax.ShapeDtypeStruct((m, n), jnp.bfloat16),
        in_specs=[pl.BlockSpec((BLK, n), lambda b: (b, 0))],
        out_specs=pl.BlockSpec((m, n), lambda b: (0, 0)),
        scratch_shapes=[
            pltpu.VMEM((1, n), jnp.float32),
            pltpu.VMEM((1, n), jnp.float32),
            pltpu.VMEM((N_ROUNDS, 1, n), jnp.float32),
            pltpu.VMEM((N_ROUNDS, 1, n), jnp.float32),
            pltpu.SemaphoreType.DMA((N_ROUNDS,)),
            pltpu.SemaphoreType.DMA((N_ROUNDS,)),
        ],
        compiler_params=pltpu.CompilerParams(collective_id=0),
    )(x)


# device time: 42748 ns/iter; 1.4584x vs baseline; 1.0569x over previous
import functools

import jax
import jax.numpy as jnp
from jax import lax
from jax.experimental import pallas as pl
from jax.experimental.pallas import tpu as pltpu

N_DEV = 8
BLK = 512


def kernel(x):
    x = x.astype(jnp.float32)
    m, n = x.shape
    nblk = m // BLK

    def body(
        x_ref, o_ref, carry_ref, comm_ref, stage_ref, send_sems, recv_sems
    ):
        b = pl.program_id(0)
        i = lax.axis_index("i")

        @pl.when(b == 0)
        def _():
            carry_ref[...] = jnp.ones_like(carry_ref)

        y = x_ref[...]
        s = 1
        while s < BLK:
            pad = jnp.ones((s, n), jnp.float32)
            y = y * jnp.concatenate([pad, y[:-s, :]], axis=0)
            s *= 2
        y = y * carry_ref[...]
        o_ref[pl.ds(b * BLK, BLK), :] = y.astype(jnp.bfloat16)
        carry_ref[...] = y[BLK - 1 : BLK, :]

        @pl.when(b == nblk - 1)
        def _():
            comm_ref[...] = jnp.ones_like(comm_ref)
            stage_ref[...] = carry_ref[...]

            barrier_sem = pltpu.get_barrier_semaphore()
            for dd in range(1, N_DEV):
                pl.semaphore_signal(
                    barrier_sem, inc=1,
                    device_id=((i + dd) % N_DEV,),
                    device_id_type=pl.DeviceIdType.MESH,
                )
            pl.semaphore_wait(barrier_sem, N_DEV - 1)

            sends = []
            for d in range(1, N_DEV):
                send = pltpu.make_async_remote_copy(
                    src_ref=stage_ref,
                    dst_ref=comm_ref.at[d - 1],
                    send_sem=send_sems.at[d - 1],
                    recv_sem=recv_sems.at[d - 1],
                    device_id=(i + d,),
                    device_id_type=pl.DeviceIdType.MESH,
                )
                sends.append(send)

                @pl.when(i + d < N_DEV)
                def _():
                    send.start()

            for d in range(1, N_DEV):
                @pl.when(i >= d)
                def _():
                    sends[d - 1].wait_recv()

            @pl.when(i > 0)
            def _():
                p = carry_ref[...]
                p = jnp.ones_like(p)
                for d in range(1, N_DEV):
                    p = p * comm_ref[d - 1]
                o_ref[...] = (o_ref[...].astype(jnp.float32) * p).astype(
                    jnp.bfloat16
                )

            for d in range(1, N_DEV):
                @pl.when(i + d < N_DEV)
                def _():
                    sends[d - 1].wait_send()

            @functools.partial(
                pl.run_scoped, exit_sem=pltpu.SemaphoreType.REGULAR
            )
            def _(exit_sem):
                for dd in range(1, N_DEV):
                    pl.semaphore_signal(
                        exit_sem, inc=1,
                        device_id=((i + dd) % N_DEV,),
                        device_id_type=pl.DeviceIdType.MESH,
                    )
                pl.semaphore_wait(exit_sem, N_DEV - 1)

    return pl.pallas_call(
        body,
        grid=(nblk,),
        out_shape=jax.ShapeDtypeStruct((m, n), jnp.bfloat16),
        in_specs=[pl.BlockSpec((BLK, n), lambda b: (b, 0))],
        out_specs=pl.BlockSpec((m, n), lambda b: (0, 0)),
        scratch_shapes=[
            pltpu.VMEM((1, n), jnp.float32),
            pltpu.VMEM((N_DEV - 1, 1, n), jnp.float32),
            pltpu.VMEM((1, n), jnp.float32),
            pltpu.SemaphoreType.DMA((N_DEV - 1,)),
            pltpu.SemaphoreType.DMA((N_DEV - 1,)),
        ],
        compiler_params=pltpu.CompilerParams(collective_id=0),
    )(x)


# device time: 38120 ns/iter; 1.6355x vs baseline; 1.1214x over previous
import jax
import jax.numpy as jnp
from jax import lax
from jax.experimental import pallas as pl
from jax.experimental.pallas import tpu as pltpu

N_DEV = 8
BLK = 512


def kernel(x):
    x = x.astype(jnp.float32)
    m, n = x.shape
    nblk = m // BLK

    def body(
        x_ref, o_ref, carry_ref, comm_ref, stage_ref, send_sems, recv_sems
    ):
        b = pl.program_id(0)
        i = lax.axis_index("i")

        @pl.when(b == 0)
        def _():
            carry_ref[...] = jnp.ones_like(carry_ref)
            barrier_sem = pltpu.get_barrier_semaphore()
            for dd in range(1, N_DEV):
                pl.semaphore_signal(
                    barrier_sem, inc=1,
                    device_id=((i + dd) % N_DEV,),
                    device_id_type=pl.DeviceIdType.MESH,
                )
            pl.semaphore_wait(barrier_sem, N_DEV - 1)

        y = x_ref[...]
        s = 1
        while s < BLK:
            pad = jnp.ones((s, n), jnp.float32)
            y = y * jnp.concatenate([pad, y[:-s, :]], axis=0)
            s *= 2
        y = y * carry_ref[...]
        o_ref[pl.ds(b * BLK, BLK), :] = y.astype(jnp.bfloat16)
        carry_ref[...] = y[BLK - 1 : BLK, :]

        @pl.when(b == nblk - 1)
        def _():
            stage_ref[...] = carry_ref[...]

            sends = []
            for d in range(1, N_DEV):
                send = pltpu.make_async_remote_copy(
                    src_ref=stage_ref,
                    dst_ref=comm_ref.at[d - 1],
                    send_sem=send_sems.at[d - 1],
                    recv_sem=recv_sems.at[d - 1],
                    device_id=(i + d,),
                    device_id_type=pl.DeviceIdType.MESH,
                )
                sends.append(send)

                @pl.when(i + d < N_DEV)
                def _():
                    send.start()

            for d in range(1, N_DEV):
                @pl.when(i >= d)
                def _():
                    sends[d - 1].wait_recv()

            @pl.when(i > 0)
            def _():
                p = jnp.ones((1, n), jnp.float32)
                for d in range(1, N_DEV):
                    p = p * jnp.where(i >= d, comm_ref[d - 1], 1.0)
                o_ref[...] = (o_ref[...].astype(jnp.float32) * p).astype(
                    jnp.bfloat16
                )

            for d in range(1, N_DEV):
                @pl.when(i + d < N_DEV)
                def _():
                    sends[d - 1].wait_send()

    return pl.pallas_call(
        body,
        grid=(nblk,),
        out_shape=jax.ShapeDtypeStruct((m, n), jnp.bfloat16),
        in_specs=[pl.BlockSpec((BLK, n), lambda b: (b, 0))],
        out_specs=pl.BlockSpec((m, n), lambda b: (0, 0)),
        scratch_shapes=[
            pltpu.VMEM((1, n), jnp.float32),
            pltpu.VMEM((N_DEV - 1, 1, n), jnp.float32),
            pltpu.VMEM((1, n), jnp.float32),
            pltpu.SemaphoreType.DMA((N_DEV - 1,)),
            pltpu.SemaphoreType.DMA((N_DEV - 1,)),
        ],
        compiler_params=pltpu.CompilerParams(collective_id=0),
    )(x)


# device time: 36812 ns/iter; 1.6936x vs baseline; 1.0355x over previous
import jax
import jax.numpy as jnp
from jax import lax
from jax.experimental import pallas as pl
from jax.experimental.pallas import tpu as pltpu

N_DEV = 8
BLK = 1024


def kernel(x):
    x = x.astype(jnp.float32)
    m, n = x.shape
    nblk = m // BLK

    def body(
        x_ref, o_ref, carry_ref, comm_ref, stage_ref, send_sems, recv_sems
    ):
        b = pl.program_id(0)
        i = lax.axis_index("i")

        @pl.when(b == 0)
        def _():
            carry_ref[...] = jnp.ones_like(carry_ref)
            barrier_sem = pltpu.get_barrier_semaphore()
            for dd in range(1, N_DEV):
                pl.semaphore_signal(
                    barrier_sem, inc=1,
                    device_id=((i + dd) % N_DEV,),
                    device_id_type=pl.DeviceIdType.MESH,
                )
            pl.semaphore_wait(barrier_sem, N_DEV - 1)

        y = x_ref[...]
        y = jnp.concatenate([y[0:1, :] * carry_ref[...], y[1:, :]], axis=0)
        s = 1
        while s < BLK:
            y = jnp.concatenate(
                [y[:s, :], y[s:, :] * y[: BLK - s, :]], axis=0
            )
            s *= 2
        o_ref[pl.ds(b * BLK, BLK), :] = y.astype(jnp.bfloat16)
        carry_ref[...] = y[BLK - 1 : BLK, :]

        @pl.when(b == nblk - 1)
        def _():
            stage_ref[...] = carry_ref[...]

            sends = []
            for d in range(1, N_DEV):
                send = pltpu.make_async_remote_copy(
                    src_ref=stage_ref,
                    dst_ref=comm_ref.at[d - 1],
                    send_sem=send_sems.at[d - 1],
                    recv_sem=recv_sems.at[d - 1],
                    device_id=(i + d,),
                    device_id_type=pl.DeviceIdType.MESH,
                )
                sends.append(send)

                @pl.when(i + d < N_DEV)
                def _():
                    send.start()

            for d in range(1, N_DEV):
                @pl.when(i >= d)
                def _():
                    sends[d - 1].wait_recv()

            @pl.when(i > 0)
            def _():
                p = jnp.ones((1, n), jnp.float32)
                for d in range(1, N_DEV):
                    p = p * jnp.where(i >= d, comm_ref[d - 1], 1.0)
                o_ref[...] = (o_ref[...].astype(jnp.float32) * p).astype(
                    jnp.bfloat16
                )

            for d in range(1, N_DEV):
                @pl.when(i + d < N_DEV)
                def _():
                    sends[d - 1].wait_send()

    return pl.pallas_call(
        body,
        grid=(nblk,),
        out_shape=jax.ShapeDtypeStruct((m, n), jnp.bfloat16),
        in_specs=[pl.BlockSpec((BLK, n), lambda b: (b, 0))],
        out_specs=pl.BlockSpec((m, n), lambda b: (0, 0)),
        scratch_shapes=[
            pltpu.VMEM((1, n), jnp.float32),
            pltpu.VMEM((N_DEV - 1, 1, n), jnp.float32),
            pltpu.VMEM((1, n), jnp.float32),
            pltpu.SemaphoreType.DMA((N_DEV - 1,)),
            pltpu.SemaphoreType.DMA((N_DEV - 1,)),
        ],
        compiler_params=pltpu.CompilerParams(collective_id=0),
    )(x)
